# baseline (speedup 1.0000x reference)
import jax
import jax.numpy as jnp
from jax.experimental import pallas as pl
from jax.experimental.pallas import tpu as pltpu


def _concat_kernel(task_id_ref, x_ref, p_ref, o_ref):
    seq = x_ref.shape[1]
    o_ref[0, :seq, :] = x_ref[0]
    o_ref[0, seq, :] = p_ref[0, 0]


def kernel(x, task_id, prompt):
    B, S, D = x.shape
    task_id32 = task_id.astype(jnp.int32)
    prompt3 = prompt.reshape(prompt.shape[0], 1, prompt.shape[1])

    grid_spec = pltpu.PrefetchScalarGridSpec(
        num_scalar_prefetch=1,
        grid=(B,),
        in_specs=[
            pl.BlockSpec((1, S, D), lambda b, tid: (b, 0, 0)),
            pl.BlockSpec((1, 1, D), lambda b, tid: (tid[b], 0, 0)),
        ],
        out_specs=pl.BlockSpec((1, S + 1, D), lambda b, tid: (b, 0, 0)),
    )

    out = pl.pallas_call(
        _concat_kernel,
        grid_spec=grid_spec,
        out_shape=jax.ShapeDtypeStruct((B, S + 1, D), x.dtype),
    )(task_id32, x, prompt3)
    return (out, task_id)
